# baseline (speedup 1.0000x reference)
import functools

import jax
import jax.numpy as jnp
from jax import lax
from jax.experimental import pallas as pl
from jax.experimental.pallas import tpu as pltpu
from jax.experimental.pallas import tpu_sc as plsc

N = 100000
C = 5
LANES = 16
NW = 32
CHUNK = 3200
CHUNK_A = 1664
CHUNK_B = CHUNK - CHUNK_A
GROUPS = CHUNK // LANES
GROUPS_A = CHUNK_A // LANES
GROUPS_B = CHUNK_B // LANES
SC_END = (N // 128) * 128
LAST_BASE = SC_END - CHUNK
LAST_G0 = ((NW - 1) * CHUNK - LAST_BASE) // LANES

TAIL_BLK = 512
TAIL_IDX = N // TAIL_BLK
TAIL_LO = SC_END - TAIL_IDX * TAIL_BLK
TAIL_HI = N - TAIL_IDX * TAIL_BLK


def _sc_partials(logits_t, targets):
    mesh = plsc.VectorSubcoreMesh(core_axis_name="c", subcore_axis_name="s")

    @functools.partial(
        pl.kernel,
        out_type=jax.ShapeDtypeStruct((NW, C, LANES), jnp.float32),
        mesh=mesh,
        scratch_types=[
            pltpu.VMEM((C, CHUNK_A), jnp.float32),
            pltpu.VMEM((C, CHUNK_B), jnp.float32),
            pltpu.VMEM((CHUNK_A,), jnp.int32),
            pltpu.VMEM((CHUNK_B,), jnp.int32),
            pltpu.VMEM((C, LANES), jnp.float32),
            pltpu.SemaphoreType.DMA,
            pltpu.SemaphoreType.DMA,
            pltpu.SemaphoreType.DMA,
            pltpu.SemaphoreType.DMA,
        ],
    )
    def sc_kernel(lg_hbm, tg_hbm, out_hbm, lg_a, lg_b, tg_a, tg_b, part_v,
                  sem_la, sem_lb, sem_ta, sem_tb):
        wid = lax.axis_index("c") * 16 + lax.axis_index("s")
        is_last = wid == NW - 1
        base = jnp.where(is_last, LAST_BASE, wid * CHUNK)
        g0 = jnp.where(is_last, LAST_G0, 0)
        g0a = jnp.minimum(g0, GROUPS_A)
        g0b = jnp.maximum(g0 - GROUPS_A, 0)
        base_b = base + CHUNK_A
        cp_la = pltpu.async_copy(
            lg_hbm.at[:, pl.ds(base, CHUNK_A)], lg_a, sem_la)
        cp_ta = pltpu.async_copy(tg_hbm.at[pl.ds(base, CHUNK_A)], tg_a, sem_ta)
        cp_lb = pltpu.async_copy(
            lg_hbm.at[:, pl.ds(base_b, CHUNK_B)], lg_b, sem_lb)
        cp_tb = pltpu.async_copy(
            tg_hbm.at[pl.ds(base_b, CHUNK_B)], tg_b, sem_tb)

        zeros = jnp.zeros((LANES,), jnp.float32)

        def group_body(lg_v, tg_v):
            def body(g, acc_in):
                s_t, s_p, s_tt, s_pp, s_tp = acc_in
                vs = [lg_v[c, pl.ds(g * LANES, LANES)] for c in range(C)]
                m = vs[0]
                for c in range(1, C):
                    m = jnp.maximum(m, vs[c])
                p = zeros
                for c in range(C - 1, -1, -1):
                    p = jnp.where(vs[c] == m, jnp.float32(c), p)
                t = tg_v[pl.ds(g * LANES, LANES)].astype(jnp.float32)
                return (s_t + t, s_p + p, s_tt + t * t, s_pp + p * p,
                        s_tp + t * p)
            return body

        cp_la.wait()
        cp_ta.wait()
        acc = plsc.parallel_loop(
            g0a, GROUPS_A, 1, unroll=4, carry=(zeros,) * 5)(
                group_body(lg_a, tg_a))
        cp_lb.wait()
        cp_tb.wait()
        acc = plsc.parallel_loop(
            g0b, GROUPS_B, 1, unroll=4, carry=acc)(group_body(lg_b, tg_b))

        for i in range(C):
            part_v[i] = acc[i]
        pltpu.sync_copy(part_v, out_hbm.at[wid])

    return sc_kernel(logits_t, targets)


def _qwk_epilogue(parts, logits_t, targets):
    def tc_body(p_ref, lt_ref, tg_ref, o_ref):
        x = p_ref[...]
        s_t = jnp.sum(x[:, 0, :])
        s_p = jnp.sum(x[:, 1, :])
        s_tt = jnp.sum(x[:, 2, :])
        s_pp = jnp.sum(x[:, 3, :])
        s_tp = jnp.sum(x[:, 4, :])

        blk = lt_ref[...]
        col = lax.broadcasted_iota(jnp.int32, (1, TAIL_BLK), 1)
        valid = (col >= TAIL_LO) & (col < TAIL_HI)
        m = blk[0:1, :]
        p = jnp.zeros((1, TAIL_BLK), jnp.float32)
        for c in range(1, C):
            vc = blk[c:c + 1, :]
            gt = vc > m
            m = jnp.where(gt, vc, m)
            p = jnp.where(gt, jnp.float32(c), p)
        t = jnp.where(valid, tg_ref[...].reshape(1, TAIL_BLK), 0)
        t = t.astype(jnp.float32)
        p = jnp.where(valid, p, 0.0)
        s_t = s_t + jnp.sum(t)
        s_p = s_p + jnp.sum(p)
        s_tt = s_tt + jnp.sum(t * t)
        s_pp = s_pp + jnp.sum(p * p)
        s_tp = s_tp + jnp.sum(t * p)

        total = jnp.float32(N)
        wnorm = jnp.float32((C - 1) ** 2)
        num = (s_tt - 2.0 * s_tp + s_pp) / wnorm
        den = (total * s_tt - 2.0 * s_t * s_p + total * s_pp) / (wnorm * total)
        o_ref[0, 0] = num / (den + jnp.float32(1e-6))

    out = pl.pallas_call(
        tc_body,
        out_shape=jax.ShapeDtypeStruct((1, 1), jnp.float32),
        grid=(1,),
        in_specs=[
            pl.BlockSpec((NW, C, LANES), lambda i: (0, 0, 0)),
            pl.BlockSpec((C, TAIL_BLK), lambda i: (0, TAIL_IDX)),
            pl.BlockSpec((TAIL_BLK,), lambda i: (TAIL_IDX,)),
        ],
        out_specs=pl.BlockSpec((1, 1), lambda i: (0, 0),
                               memory_space=pltpu.SMEM),
    )(parts, logits_t, targets)
    return out[0, 0]


def kernel(logits, targets):
    logits_t = logits.T
    parts = _sc_partials(logits_t, targets)
    return _qwk_epilogue(parts, logits_t, targets)


# speedup vs baseline: 1.0059x; 1.0059x over previous
import functools

import jax
import jax.numpy as jnp
from jax import lax
from jax.experimental import pallas as pl
from jax.experimental.pallas import tpu as pltpu
from jax.experimental.pallas import tpu_sc as plsc

N = 100000
C = 5
LANES = 16
NW = 32
CHUNK = 3200
GROUPS = CHUNK // LANES
SC_END = (N // 128) * 128
LAST_BASE = SC_END - CHUNK
LAST_G0 = ((NW - 1) * CHUNK - LAST_BASE) // LANES

TAIL_BLK = 512
TAIL_IDX = N // TAIL_BLK
TAIL_LO = SC_END - TAIL_IDX * TAIL_BLK
TAIL_HI = N - TAIL_IDX * TAIL_BLK


def _sc_partials(logits_t, targets):
    mesh = plsc.VectorSubcoreMesh(core_axis_name="c", subcore_axis_name="s")

    @functools.partial(
        pl.kernel,
        out_type=jax.ShapeDtypeStruct((NW, C, LANES), jnp.float32),
        mesh=mesh,
        scratch_types=[
            pltpu.VMEM((C, CHUNK), jnp.float32),
            pltpu.VMEM((CHUNK,), jnp.int32),
            pltpu.VMEM((C, LANES), jnp.float32),
            pltpu.SemaphoreType.DMA,
            pltpu.SemaphoreType.DMA,
        ],
    )
    def sc_kernel(lg_hbm, tg_hbm, out_hbm, lg_v, tg_v, part_v, sem_l, sem_t):
        wid = lax.axis_index("c") * 16 + lax.axis_index("s")
        is_last = wid == NW - 1
        base = jnp.where(is_last, LAST_BASE, wid * CHUNK)
        g0 = jnp.where(is_last, LAST_G0, 0)
        cp_l = pltpu.async_copy(lg_hbm.at[:, pl.ds(base, CHUNK)], lg_v, sem_l)
        cp_t = pltpu.async_copy(tg_hbm.at[pl.ds(base, CHUNK)], tg_v, sem_t)
        cp_l.wait()
        cp_t.wait()

        zeros = jnp.zeros((LANES,), jnp.float32)

        @plsc.parallel_loop(g0, GROUPS, 1, unroll=8, carry=(zeros,) * 5)
        def acc(g, acc_in):
            s_t, s_p, s_tt, s_pp, s_tp = acc_in
            vs = [lg_v[c, pl.ds(g * LANES, LANES)] for c in range(C)]
            m = vs[0]
            for c in range(1, C):
                m = jnp.maximum(m, vs[c])
            p = zeros
            for c in range(C - 1, -1, -1):
                p = jnp.where(vs[c] == m, jnp.float32(c), p)
            t = tg_v[pl.ds(g * LANES, LANES)].astype(jnp.float32)
            return (s_t + t, s_p + p, s_tt + t * t, s_pp + p * p,
                    s_tp + t * p)

        for i in range(C):
            part_v[i] = acc[i]
        pltpu.sync_copy(part_v, out_hbm.at[wid])

    return sc_kernel(logits_t, targets)


def _qwk_epilogue(parts, logits_t, targets):
    def tc_body(p_ref, lt_ref, tg_ref, o_ref):
        x = p_ref[...]
        s_t = jnp.sum(x[:, 0, :])
        s_p = jnp.sum(x[:, 1, :])
        s_tt = jnp.sum(x[:, 2, :])
        s_pp = jnp.sum(x[:, 3, :])
        s_tp = jnp.sum(x[:, 4, :])

        blk = lt_ref[...]
        col = lax.broadcasted_iota(jnp.int32, (1, TAIL_BLK), 1)
        valid = (col >= TAIL_LO) & (col < TAIL_HI)
        m = blk[0:1, :]
        p = jnp.zeros((1, TAIL_BLK), jnp.float32)
        for c in range(1, C):
            vc = blk[c:c + 1, :]
            gt = vc > m
            m = jnp.where(gt, vc, m)
            p = jnp.where(gt, jnp.float32(c), p)
        t = jnp.where(valid, tg_ref[...].reshape(1, TAIL_BLK), 0)
        t = t.astype(jnp.float32)
        p = jnp.where(valid, p, 0.0)
        s_t = s_t + jnp.sum(t)
        s_p = s_p + jnp.sum(p)
        s_tt = s_tt + jnp.sum(t * t)
        s_pp = s_pp + jnp.sum(p * p)
        s_tp = s_tp + jnp.sum(t * p)

        total = jnp.float32(N)
        wnorm = jnp.float32((C - 1) ** 2)
        num = (s_tt - 2.0 * s_tp + s_pp) / wnorm
        den = (total * s_tt - 2.0 * s_t * s_p + total * s_pp) / (wnorm * total)
        o_ref[0, 0] = num / (den + jnp.float32(1e-6))

    out = pl.pallas_call(
        tc_body,
        out_shape=jax.ShapeDtypeStruct((1, 1), jnp.float32),
        grid=(1,),
        in_specs=[
            pl.BlockSpec((NW, C, LANES), lambda i: (0, 0, 0)),
            pl.BlockSpec((C, TAIL_BLK), lambda i: (0, TAIL_IDX)),
            pl.BlockSpec((TAIL_BLK,), lambda i: (TAIL_IDX,)),
        ],
        out_specs=pl.BlockSpec((1, 1), lambda i: (0, 0),
                               memory_space=pltpu.SMEM),
    )(parts, logits_t, targets)
    return out[0, 0]


def kernel(logits, targets):
    logits_t = logits.T
    parts = _sc_partials(logits_t, targets)
    return _qwk_epilogue(parts, logits_t, targets)
